# baseline (speedup 1.0000x reference)
import functools

import jax
import jax.numpy as jnp
from jax.experimental import pallas as pl
from jax.experimental.pallas import tpu as pltpu

_EPS = 1e-5


def _g(i, bx, rf, h1, h2, h3, w1_ref, b1_ref):
    b1i = b1_ref[i : i + 1, :]
    if i == 0:
        w1a = w1_ref[0, :128, :]
        w1b = w1_ref[0, 128:, :]
        pre = (
            jnp.dot(bx[...], w1a, preferred_element_type=jnp.float32)
            + jnp.dot(rf[...], w1b, preferred_element_type=jnp.float32)
            + b1i
        )
    else:
        f = (h1, h2, h3)[i - 1][...]
        pre = jnp.dot(f, w1_ref[i], preferred_element_type=jnp.float32) + b1i
    return jnp.maximum(pre, 0.0)


def _stats_kernel(n_rows, n_blocks,
                  bx, rf, h1, h2, h3, w1, b1, w2, b2, gamma, beta,
                  w2s_out, csum_out, sums, gram):
    step = pl.program_id(0)

    @pl.when(step == 0)
    def _init():
        sums[...] = jnp.zeros_like(sums)
        gram[...] = jnp.zeros_like(gram)

    for i in range(4):
        g = _g(i, bx, rf, h1, h2, h3, w1, b1)
        sums[i, :] = sums[i, :] + jnp.sum(g, axis=0)
        gram[i] = gram[i] + jax.lax.dot_general(
            g, g, (((0,), (0,)), ((), ())), preferred_element_type=jnp.float32)

    @pl.when(step == n_blocks - 1)
    def _finalize():
        n = jnp.float32(n_rows)
        c_total = jnp.zeros((1, 128), dtype=jnp.float32)
        for i in range(4):
            w2i = w2[i]
            b2i = b2[i : i + 1, :]
            sg = sums[i : i + 1, :]
            sw = jnp.dot(sg, w2i, preferred_element_type=jnp.float32)
            t = jnp.dot(gram[i], w2i, preferred_element_type=jnp.float32)
            q = jnp.sum(w2i * t, axis=0, keepdims=True)
            sum_h = sw + n * b2i
            sum_h2 = q + 2.0 * b2i * sw + n * b2i * b2i
            mu = sum_h / n
            var = sum_h2 / n - mu * mu
            a = gamma[i : i + 1, :] * jax.lax.rsqrt(var + _EPS)
            w2s_out[i] = w2i * a
            c_total = c_total + (b2i - mu) * a + beta[i : i + 1, :]
        csum_out[...] = c_total


def _out_kernel(bx, rf, h1, h2, h3, w1, b1, w2s, csum, out_ref):
    acc = jnp.broadcast_to(csum[...], out_ref.shape)
    for i in range(4):
        g = _g(i, bx, rf, h1, h2, h3, w1, b1)
        acc = acc + jnp.dot(g, w2s[i], preferred_element_type=jnp.float32)
    out_ref[...] = acc


def _pick_block(n):
    for b in (4000, 2000, 1000, 200, 40, 8):
        if n % b == 0:
            return b
    return n


@functools.partial(jax.jit, static_argnames=())
def kernel(x, edges, batch_x, rand_feature, hop1_feature, hop2_feature,
           hop3_feature, W1, b1, W2, b2, gamma, beta):
    del x, edges
    n, d_in = batch_x.shape
    rw = rand_feature.shape[1]
    h2_dim = W2.shape[-1]
    blk = _pick_block(n)
    n_blocks = n // blk

    row_specs = [
        pl.BlockSpec((blk, d_in), lambda i: (i, 0)),
        pl.BlockSpec((blk, rw), lambda i: (i, 0)),
        pl.BlockSpec((blk, d_in + rw), lambda i: (i, 0)),
        pl.BlockSpec((blk, d_in + rw), lambda i: (i, 0)),
        pl.BlockSpec((blk, d_in + rw), lambda i: (i, 0)),
    ]
    w1_spec = pl.BlockSpec(W1.shape, lambda i: (0, 0, 0))
    vec_spec = pl.BlockSpec((4, h2_dim), lambda i: (0, 0))
    w2_spec = pl.BlockSpec(W2.shape, lambda i: (0, 0, 0))

    w2s, csum = pl.pallas_call(
        functools.partial(_stats_kernel, n, n_blocks),
        grid=(n_blocks,),
        in_specs=row_specs + [w1_spec, vec_spec, w2_spec, vec_spec,
                              vec_spec, vec_spec],
        out_specs=[
            pl.BlockSpec(W2.shape, lambda i: (0, 0, 0)),
            pl.BlockSpec((1, h2_dim), lambda i: (0, 0)),
        ],
        out_shape=[
            jax.ShapeDtypeStruct(W2.shape, jnp.float32),
            jax.ShapeDtypeStruct((1, h2_dim), jnp.float32),
        ],
        scratch_shapes=[
            pltpu.VMEM((4, h2_dim), jnp.float32),
            pltpu.VMEM((4, h2_dim, h2_dim), jnp.float32),
        ],
        compiler_params=pltpu.CompilerParams(
            dimension_semantics=("arbitrary",)),
    )(batch_x, rand_feature, hop1_feature, hop2_feature, hop3_feature,
      W1, b1, W2, b2, gamma, beta)

    out = pl.pallas_call(
        _out_kernel,
        grid=(n_blocks,),
        in_specs=row_specs + [
            w1_spec, vec_spec, w2_spec,
            pl.BlockSpec((1, h2_dim), lambda i: (0, 0)),
        ],
        out_specs=pl.BlockSpec((blk, h2_dim), lambda i: (i, 0)),
        out_shape=jax.ShapeDtypeStruct((n, h2_dim), jnp.float32),
        compiler_params=pltpu.CompilerParams(
            dimension_semantics=("arbitrary",)),
    )(batch_x, rand_feature, hop1_feature, hop2_feature, hop3_feature,
      W1, b1, w2s, csum)

    return out


# speedup vs baseline: 1.1403x; 1.1403x over previous
import functools

import jax
import jax.numpy as jnp
from jax.experimental import pallas as pl
from jax.experimental.pallas import tpu as pltpu

_EPS = 1e-5


def _h(i, bx, rf, h1, h2, h3, w1_ref, b1_ref, w2_ref, b2_ref):
    b1i = b1_ref[i : i + 1, :]
    if i == 0:
        w1a = w1_ref[0, :128, :].astype(jnp.bfloat16)
        w1b = w1_ref[0, 128:, :].astype(jnp.bfloat16)
        pre = (
            jnp.dot(bx[...].astype(jnp.bfloat16), w1a,
                    preferred_element_type=jnp.float32)
            + jnp.dot(rf[...].astype(jnp.bfloat16), w1b,
                      preferred_element_type=jnp.float32)
            + b1i
        )
    else:
        f = (h1, h2, h3)[i - 1][...].astype(jnp.bfloat16)
        pre = jnp.dot(f, w1_ref[i].astype(jnp.bfloat16),
                      preferred_element_type=jnp.float32) + b1i
    g = jnp.maximum(pre, 0.0).astype(jnp.bfloat16)
    return jnp.dot(g, w2_ref[i].astype(jnp.bfloat16),
                   preferred_element_type=jnp.float32) + b2_ref[i : i + 1, :]


def _pass1_kernel(n_rows, n_blocks,
                  bx, rf, h1, h2, h3, w1, b1, w2, b2, gamma, beta,
                  hstore, avec_out, cvec_out, s1, s2):
    step = pl.program_id(0)

    @pl.when(step == 0)
    def _init():
        s1[...] = jnp.zeros_like(s1)
        s2[...] = jnp.zeros_like(s2)

    for i in range(4):
        h = _h(i, bx, rf, h1, h2, h3, w1, b1, w2, b2)
        hstore[i] = h.astype(jnp.bfloat16)
        s1[i, :] = s1[i, :] + jnp.sum(h, axis=0)
        s2[i, :] = s2[i, :] + jnp.sum(h * h, axis=0)

    @pl.when(step == n_blocks - 1)
    def _finalize():
        n = jnp.float32(n_rows)
        mu = s1[...] / n
        var = s2[...] / n - mu * mu
        a = gamma[...] * jax.lax.rsqrt(var + _EPS)
        avec_out[...] = a
        cvec_out[...] = jnp.sum(beta[...] - mu * a, axis=0, keepdims=True)


def _pass2_kernel(hstore, avec, cvec, out_ref):
    a = avec[...]
    acc = jnp.broadcast_to(cvec[...], out_ref.shape)
    for i in range(4):
        acc = acc + hstore[i].astype(jnp.float32) * a[i : i + 1, :]
    out_ref[...] = acc


def _pick_block(n):
    for b in (4000, 2000, 1000, 200, 40, 8):
        if n % b == 0:
            return b
    return n


@jax.jit
def kernel(x, edges, batch_x, rand_feature, hop1_feature, hop2_feature,
           hop3_feature, W1, b1, W2, b2, gamma, beta):
    del x, edges
    n, d_in = batch_x.shape
    rw = rand_feature.shape[1]
    h2_dim = W2.shape[-1]
    blk = _pick_block(n)
    n_blocks = n // blk

    row_specs = [
        pl.BlockSpec((blk, d_in), lambda i: (i, 0)),
        pl.BlockSpec((blk, rw), lambda i: (i, 0)),
        pl.BlockSpec((blk, d_in + rw), lambda i: (i, 0)),
        pl.BlockSpec((blk, d_in + rw), lambda i: (i, 0)),
        pl.BlockSpec((blk, d_in + rw), lambda i: (i, 0)),
    ]
    w1_spec = pl.BlockSpec(W1.shape, lambda i: (0, 0, 0))
    vec_spec = pl.BlockSpec((4, h2_dim), lambda i: (0, 0))
    w2_spec = pl.BlockSpec(W2.shape, lambda i: (0, 0, 0))
    hstore_spec = pl.BlockSpec((4, blk, h2_dim), lambda i: (0, i, 0))

    hstore, avec, cvec = pl.pallas_call(
        functools.partial(_pass1_kernel, n, n_blocks),
        grid=(n_blocks,),
        in_specs=row_specs + [w1_spec, vec_spec, w2_spec, vec_spec,
                              vec_spec, vec_spec],
        out_specs=[
            hstore_spec,
            pl.BlockSpec((4, h2_dim), lambda i: (0, 0)),
            pl.BlockSpec((1, h2_dim), lambda i: (0, 0)),
        ],
        out_shape=[
            jax.ShapeDtypeStruct((4, n, h2_dim), jnp.bfloat16),
            jax.ShapeDtypeStruct((4, h2_dim), jnp.float32),
            jax.ShapeDtypeStruct((1, h2_dim), jnp.float32),
        ],
        scratch_shapes=[
            pltpu.VMEM((4, h2_dim), jnp.float32),
            pltpu.VMEM((4, h2_dim), jnp.float32),
        ],
        compiler_params=pltpu.CompilerParams(
            dimension_semantics=("arbitrary",)),
    )(batch_x, rand_feature, hop1_feature, hop2_feature, hop3_feature,
      W1, b1, W2, b2, gamma, beta)

    out = pl.pallas_call(
        _pass2_kernel,
        grid=(n_blocks,),
        in_specs=[
            hstore_spec,
            pl.BlockSpec((4, h2_dim), lambda i: (0, 0)),
            pl.BlockSpec((1, h2_dim), lambda i: (0, 0)),
        ],
        out_specs=pl.BlockSpec((blk, h2_dim), lambda i: (i, 0)),
        out_shape=jax.ShapeDtypeStruct((n, h2_dim), jnp.float32),
        compiler_params=pltpu.CompilerParams(
            dimension_semantics=("arbitrary",)),
    )(hstore, avec, cvec)

    return out
